# Initial kernel scaffold; baseline (speedup 1.0000x reference)
import jax
import jax.numpy as jnp
from jax.experimental import pallas as pl


def kernel(x, edge_index, packed_idx, mask, B, temp):
    raise NotImplementedError("write your pallas kernel here")


# speedup vs baseline: 4.3212x; 4.3212x over previous
import functools

import jax
import jax.numpy as jnp
from jax import lax
from jax.experimental import pallas as pl
from jax.experimental.pallas import tpu as pltpu
from jax.experimental.pallas import tpu_sc as plsc

N = 10000
F = 128
L = 32
M = 8
K = 16
MK = M * K

NC = 2
NS = 16
NW = NC * NS

NPW = 320
N2 = NW * NPW
CH_NODES = 4
CH_ROWS = CH_NODES * L
NCH = NPW // CH_NODES
NPAIR = NCH // 2

_EPS = 1e-12


def _table_body(t_ref, x_ref, w_ref, out_ref):
    t = t_ref[0, 0]
    x = x_ref[:]
    xn = x / jnp.maximum(jnp.sqrt(jnp.sum(x * x, axis=1, keepdims=True)), _EPS)
    w = w_ref[:]
    wn = w / jnp.maximum(jnp.sqrt(jnp.sum(w * w, axis=1, keepdims=True)), _EPS)
    p = jnp.dot(xn, wn.T, preferred_element_type=jnp.float32) * t
    e = jnp.exp(p)
    r = lax.broadcasted_iota(jnp.int32, (MK, MK), 0)
    c = lax.broadcasted_iota(jnp.int32, (MK, MK), 1)
    bd = ((r // K) == (c // K)).astype(jnp.float32)
    s = jnp.dot(e, bd, preferred_element_type=jnp.float32,
                precision=lax.Precision.HIGHEST)
    out_ref[:] = e / s


def _build_table(x, mask_flat, temp):
    TN = 400
    return pl.pallas_call(
        _table_body,
        grid=(N // TN,),
        in_specs=[
            pl.BlockSpec(memory_space=pltpu.SMEM),
            pl.BlockSpec((TN, F), lambda i: (i, 0)),
            pl.BlockSpec((MK, F), lambda i: (0, 0)),
        ],
        out_specs=pl.BlockSpec((TN, MK), lambda i: (i, 0)),
        out_shape=jax.ShapeDtypeStruct((N, MK), jnp.float32),
    )(jnp.reshape(jnp.abs(temp), (1, 1)), x, mask_flat)


def _fire(table_hbm, idx_v, rows, sem, ch):
    return pltpu.async_copy(
        table_hbm.at[idx_v.at[pl.ds(ch * CH_ROWS, CH_ROWS)]], rows, sem)


def _wait(table_hbm, idx_v, rows, sem, ch):
    pltpu.make_async_copy(
        table_hbm.at[idx_v.at[pl.ds(ch * CH_ROWS, CH_ROWS)]], rows, sem).wait()


def _acc_chunk(rows, acc_v, ch):
    for c4 in range(CH_NODES):
        base_r = c4 * L

        def lbody(i, accs, base_r=base_r):
            r0 = base_r + 2 * i
            out = []
            for j in range(8):
                a = accs[j]
                a = a + rows[r0, pl.ds(16 * j, 16)]
                a = a + rows[r0 + 1, pl.ds(16 * j, 16)]
                out.append(a)
            return tuple(out)

        z = jnp.zeros((16,), jnp.float32)
        accs = lax.fori_loop(0, L // 2, lbody, (z,) * 8)
        node = ch * CH_NODES + c4
        for j in range(8):
            acc_v[node, pl.ds(16 * j, 16)] = accs[j]


def _gather_sum_body(table_hbm, pidx_hbm, out_hbm,
                     idx_v, rows0, rows1, acc_v, sem0, sem1):
    cid = lax.axis_index("c")
    sid = lax.axis_index("s")
    wid = sid * NC + cid
    base = wid * NPW
    pltpu.sync_copy(pidx_hbm.at[pl.ds(base * L, NPW * L)], idx_v)
    _fire(table_hbm, idx_v, rows0, sem0, 0)

    def obody(o, carry):
        ch0 = 2 * o
        ch1 = ch0 + 1
        _fire(table_hbm, idx_v, rows1, sem1, ch1)
        _wait(table_hbm, idx_v, rows0, sem0, ch0)
        _acc_chunk(rows0, acc_v, ch0)

        @pl.when(o < NPAIR - 1)
        def _():
            _fire(table_hbm, idx_v, rows0, sem0, ch0 + 2)

        _wait(table_hbm, idx_v, rows1, sem1, ch1)
        _acc_chunk(rows1, acc_v, ch1)
        return carry

    lax.fori_loop(0, NPAIR, obody, 0)
    pltpu.sync_copy(acc_v, out_hbm.at[pl.ds(base, NPW)])


def _gather_sum(table_full, pidx_flat):
    mesh = plsc.VectorSubcoreMesh(
        core_axis_name="c", subcore_axis_name="s",
        num_cores=NC, num_subcores=NS)
    return pl.kernel(
        _gather_sum_body,
        out_type=jax.ShapeDtypeStruct((N2, MK), jnp.float32),
        mesh=mesh,
        scratch_types=[
            pltpu.VMEM((NPW * L,), jnp.int32),
            pltpu.VMEM((CH_ROWS, MK), jnp.float32),
            pltpu.VMEM((CH_ROWS, MK), jnp.float32),
            pltpu.VMEM((NPW, MK), jnp.float32),
            pltpu.SemaphoreType.DMA,
            pltpu.SemaphoreType.DMA,
        ],
    )(table_full, pidx_flat)


def _final_body(a_ref, b_ref, out_ref):
    d = jnp.abs(a_ref[:] - b_ref[:])
    r = lax.broadcasted_iota(jnp.int32, (MK, M), 0)
    c = lax.broadcasted_iota(jnp.int32, (MK, M), 1)
    g = ((r // K) == c).astype(jnp.float32)
    out_ref[:] = 1.0 - 0.5 * jnp.dot(d, g, preferred_element_type=jnp.float32,
                                     precision=lax.Precision.HIGHEST)


def _finalize(acc, b_flat):
    TN = 512
    return pl.pallas_call(
        _final_body,
        grid=(N2 // TN,),
        in_specs=[
            pl.BlockSpec((TN, MK), lambda i: (i, 0)),
            pl.BlockSpec((1, MK), lambda i: (0, 0)),
        ],
        out_specs=pl.BlockSpec((TN, M), lambda i: (i, 0)),
        out_shape=jax.ShapeDtypeStruct((N2, M), jnp.float32),
    )(acc, b_flat)


def kernel(x, edge_index, packed_idx, mask, B, temp):
    del edge_index
    table = _build_table(x, jnp.reshape(mask, (MK, F)), temp)
    table_full = jnp.concatenate(
        [jnp.zeros((1, MK), jnp.float32), table], axis=0)
    pidx = jnp.pad(packed_idx, ((0, N2 - N), (0, 0)))
    acc = _gather_sum(table_full, jnp.reshape(pidx, (-1,)))
    res = _finalize(acc, jnp.reshape(B, (1, MK)))
    return res[:N]


# speedup vs baseline: 15.5147x; 3.5904x over previous
import functools

import jax
import jax.numpy as jnp
from jax import lax
from jax.experimental import pallas as pl
from jax.experimental.pallas import tpu as pltpu
from jax.experimental.pallas import tpu_sc as plsc

N = 10000
F = 128
L = 32
M = 8
K = 16
MK = M * K

NC = 2
NS = 16
NW = NC * NS

NPW = 320
N2 = NW * NPW
CH_NODES = 4
CH_ROWS = CH_NODES * L
NCH = NPW // CH_NODES
NPAIR = NCH // 2
TPS = 8 * ((N + 1 + 8 * NS - 1) // (8 * NS))
NT = NS * TPS

_EPS = 1e-12


def _table_body(t_ref, x_ref, w_ref, out_ref):
    t = t_ref[0, 0]
    x = x_ref[:]
    xn = x / jnp.maximum(jnp.sqrt(jnp.sum(x * x, axis=1, keepdims=True)), _EPS)
    w = w_ref[:]
    wn = w / jnp.maximum(jnp.sqrt(jnp.sum(w * w, axis=1, keepdims=True)), _EPS)
    p = jnp.dot(xn, wn.T, preferred_element_type=jnp.float32) * t
    e = jnp.exp(p)
    r = lax.broadcasted_iota(jnp.int32, (MK, MK), 0)
    c = lax.broadcasted_iota(jnp.int32, (MK, MK), 1)
    bd = ((r // K) == (c // K)).astype(jnp.float32)
    s = jnp.dot(e, bd, preferred_element_type=jnp.float32,
                precision=lax.Precision.HIGHEST)
    out_ref[:] = e / s


def _build_table(x, mask_flat, temp):
    TN = 400
    return pl.pallas_call(
        _table_body,
        grid=(N // TN,),
        in_specs=[
            pl.BlockSpec(memory_space=pltpu.SMEM),
            pl.BlockSpec((TN, F), lambda i: (i, 0)),
            pl.BlockSpec((MK, F), lambda i: (0, 0)),
        ],
        out_specs=pl.BlockSpec((TN, MK), lambda i: (i, 0)),
        out_shape=jax.ShapeDtypeStruct((N, MK), jnp.float32),
    )(jnp.reshape(jnp.abs(temp), (1, 1)), x, mask_flat)


def _fire(table_hbm, idx_v, rows, sem, ch):
    return pltpu.async_copy(
        table_hbm.at[idx_v.at[pl.ds(ch * CH_ROWS, CH_ROWS)]], rows, sem)


def _wait(table_hbm, idx_v, rows, sem, ch):
    pltpu.make_async_copy(
        table_hbm.at[idx_v.at[pl.ds(ch * CH_ROWS, CH_ROWS)]], rows, sem).wait()


GCH = 8
NG = NCH // GCH
GN = GCH * CH_NODES


def _acc_chunk(rows, acc_v, k):
    for c4 in range(CH_NODES):
        base_r = c4 * L

        def lbody(i, accs, base_r=base_r):
            r0 = base_r + 2 * i
            out = []
            for j in range(8):
                a = accs[j]
                a = a + rows[r0, pl.ds(16 * j, 16)]
                a = a + rows[r0 + 1, pl.ds(16 * j, 16)]
                out.append(a)
            return tuple(out)

        z = jnp.zeros((16,), jnp.float32)
        accs = lax.fori_loop(0, L // 2, lbody, (z,) * 8)
        node = k * CH_NODES + c4
        for j in range(8):
            acc_v[node, pl.ds(16 * j, 16)] = accs[j]


def _gather_sum_body(table_hbm, pidx_hbm, out_hbm,
                     idx_v, rows0, rows1, acc_v, tab_sh, sem0, sem1):
    cid = lax.axis_index("c")
    sid = lax.axis_index("s")
    wid = sid * NC + cid
    base = wid * NPW
    pltpu.sync_copy(table_hbm.at[pl.ds(sid * TPS, TPS)],
                    tab_sh.at[pl.ds(sid * TPS, TPS)])
    pltpu.sync_copy(pidx_hbm.at[pl.ds(base * L, NPW * L)], idx_v)
    plsc.subcore_barrier()
    _fire(tab_sh, idx_v, rows0, sem0, 0)

    def obody(g, carry):
        for k in range(GCH):
            ch = g * GCH + k
            rows, sem = (rows0, sem0) if k % 2 == 0 else (rows1, sem1)
            nrows, nsem = (rows1, sem1) if k % 2 == 0 else (rows0, sem0)

            @pl.when(ch < NCH - 1)
            def _(ch=ch, nrows=nrows, nsem=nsem):
                _fire(tab_sh, idx_v, nrows, nsem, ch + 1)

            _wait(tab_sh, idx_v, rows, sem, ch)
            _acc_chunk(rows, acc_v, k)
        pltpu.sync_copy(acc_v, out_hbm.at[pl.ds(base + g * GN, GN)])
        return carry

    lax.fori_loop(0, NG, obody, 0)


def _gather_sum(table_full, pidx_flat):
    mesh = plsc.VectorSubcoreMesh(
        core_axis_name="c", subcore_axis_name="s",
        num_cores=NC, num_subcores=NS)
    return pl.kernel(
        _gather_sum_body,
        out_type=jax.ShapeDtypeStruct((N2, MK), jnp.float32),
        mesh=mesh,
        scratch_types=[
            pltpu.VMEM((NPW * L,), jnp.int32),
            pltpu.VMEM((CH_ROWS, MK), jnp.float32),
            pltpu.VMEM((CH_ROWS, MK), jnp.float32),
            pltpu.VMEM((GN, MK), jnp.float32),
            pltpu.VMEM_SHARED((NT, MK), jnp.float32),
            pltpu.SemaphoreType.DMA,
            pltpu.SemaphoreType.DMA,
        ],
    )(table_full, pidx_flat)


def _final_body(a_ref, b_ref, out_ref):
    d = jnp.abs(a_ref[:] - b_ref[:])
    r = lax.broadcasted_iota(jnp.int32, (MK, M), 0)
    c = lax.broadcasted_iota(jnp.int32, (MK, M), 1)
    g = ((r // K) == c).astype(jnp.float32)
    out_ref[:] = 1.0 - 0.5 * jnp.dot(d, g, preferred_element_type=jnp.float32,
                                     precision=lax.Precision.HIGHEST)


def _finalize(acc, b_flat):
    TN = 512
    return pl.pallas_call(
        _final_body,
        grid=(N2 // TN,),
        in_specs=[
            pl.BlockSpec((TN, MK), lambda i: (i, 0)),
            pl.BlockSpec((1, MK), lambda i: (0, 0)),
        ],
        out_specs=pl.BlockSpec((TN, M), lambda i: (i, 0)),
        out_shape=jax.ShapeDtypeStruct((N2, M), jnp.float32),
    )(acc, b_flat)


def kernel(x, edge_index, packed_idx, mask, B, temp):
    del edge_index
    table = _build_table(x, jnp.reshape(mask, (MK, F)), temp)
    table_full = jnp.concatenate(
        [jnp.zeros((1, MK), jnp.float32), table,
         jnp.zeros((NT - N - 1, MK), jnp.float32)], axis=0)
    pidx = jnp.pad(packed_idx, ((0, N2 - N), (0, 0)))
    acc = _gather_sum(table_full, jnp.reshape(pidx, (-1,)))
    res = _finalize(acc, jnp.reshape(B, (1, MK)))
    return res[:N]


# speedup vs baseline: 22.2846x; 1.4364x over previous
import functools

import jax
import jax.numpy as jnp
from jax import lax
from jax.experimental import pallas as pl
from jax.experimental.pallas import tpu as pltpu
from jax.experimental.pallas import tpu_sc as plsc

N = 10000
F = 128
L = 32
M = 8
K = 16
MK = M * K

NC = 2
NS = 16
NW = NC * NS

NPW = 320
N2 = NW * NPW
CH_NODES = 4
CH_ROWS = CH_NODES * L
NCH = NPW // CH_NODES
GCH = 8
NG = NCH // GCH
GN = GCH * CH_NODES
TPS = 16 * ((N + 1 + 16 * NS - 1) // (16 * NS))
NT = NS * TPS

_EPS = 1e-12


def _table_body(t_ref, x_ref, w_ref, out_ref):
    i = pl.program_id(0)
    t = t_ref[0, 0]
    x = x_ref[:]
    xn = x / jnp.maximum(jnp.sqrt(jnp.sum(x * x, axis=1, keepdims=True)), _EPS)
    w = w_ref[:]
    wn = w / jnp.maximum(jnp.sqrt(jnp.sum(w * w, axis=1, keepdims=True)), _EPS)
    p = jnp.dot(xn, wn.T, preferred_element_type=jnp.float32) * t
    e = jnp.exp(p)
    r = lax.broadcasted_iota(jnp.int32, (MK, MK), 0)
    c = lax.broadcasted_iota(jnp.int32, (MK, MK), 1)
    bd = ((r // K) == (c // K)).astype(jnp.float32)
    s = jnp.dot(e, bd, preferred_element_type=jnp.float32,
                precision=lax.Precision.HIGHEST)
    sm = e / s
    TN = out_ref.shape[0]
    row = i * TN + lax.broadcasted_iota(jnp.int32, sm.shape, 0)
    smz = jnp.where(row < N, sm, 0.0).astype(jnp.bfloat16)
    lo = lax.bitcast_convert_type(smz[:, :64], jnp.uint16).astype(jnp.uint32)
    hi = lax.bitcast_convert_type(smz[:, 64:], jnp.uint16).astype(jnp.uint32)
    out_ref[:] = (lo | (hi << 16)).astype(jnp.int32)


def _build_table(x_pad, mask_flat, temp):
    TN = 640
    return pl.pallas_call(
        _table_body,
        grid=(NT // TN,),
        in_specs=[
            pl.BlockSpec(memory_space=pltpu.SMEM),
            pl.BlockSpec((TN, F), lambda i: (i, 0)),
            pl.BlockSpec((MK, F), lambda i: (0, 0)),
        ],
        out_specs=pl.BlockSpec((TN, MK // 2), lambda i: (i, 0)),
        out_shape=jax.ShapeDtypeStruct((NT, MK // 2), jnp.int32),
    )(jnp.reshape(jnp.abs(temp), (1, 1)), x_pad, mask_flat)


def _fire(tab, idx_v, rows, sem, ch):
    return pltpu.async_copy(
        tab.at[idx_v.at[pl.ds(ch * CH_ROWS, CH_ROWS)]], rows, sem)


def _wait(tab, idx_v, rows, sem, ch):
    pltpu.make_async_copy(
        tab.at[idx_v.at[pl.ds(ch * CH_ROWS, CH_ROWS)]], rows, sem).wait()


def _acc_chunk(rows, acc_v, k):
    for c4 in range(CH_NODES):
        base_r = c4 * L

        def lbody(i, accs, base_r=base_r):
            r0 = base_r + 2 * i
            los, his = list(accs[:4]), list(accs[4:])
            for j in range(4):
                w0 = rows[r0, pl.ds(16 * j, 16)]
                w1 = rows[r0 + 1, pl.ds(16 * j, 16)]
                his[j] = (his[j] + lax.bitcast_convert_type(w0, jnp.float32)
                          + lax.bitcast_convert_type(w1, jnp.float32))
                los[j] = (los[j]
                          + lax.bitcast_convert_type(w0 << 16, jnp.float32)
                          + lax.bitcast_convert_type(w1 << 16, jnp.float32))
            return tuple(los) + tuple(his)

        z = jnp.zeros((16,), jnp.float32)
        accs = lax.fori_loop(0, L // 2, lbody, (z,) * 8)
        node = k * CH_NODES + c4
        for j in range(4):
            acc_v[node, pl.ds(16 * j, 16)] = accs[j]
            acc_v[node, pl.ds(64 + 16 * j, 16)] = accs[4 + j]


def _gather_sum_body(table_hbm, pidx_hbm, out_hbm,
                     idx_v, rows0, rows1, acc_v, tab_sh, sem0, sem1):
    cid = lax.axis_index("c")
    sid = lax.axis_index("s")
    wid = sid * NC + cid
    base = wid * NPW
    pltpu.sync_copy(table_hbm.at[pl.ds(sid * TPS, TPS)],
                    tab_sh.at[pl.ds(sid * TPS, TPS)])
    pltpu.sync_copy(pidx_hbm.at[pl.ds(base * L, NPW * L)], idx_v)
    plsc.subcore_barrier()
    _fire(tab_sh, idx_v, rows0, sem0, 0)

    def obody(g, carry):
        for k in range(GCH):
            ch = g * GCH + k
            rows, sem = (rows0, sem0) if k % 2 == 0 else (rows1, sem1)
            nrows, nsem = (rows1, sem1) if k % 2 == 0 else (rows0, sem0)

            @pl.when(ch < NCH - 1)
            def _(ch=ch, nrows=nrows, nsem=nsem):
                _fire(tab_sh, idx_v, nrows, nsem, ch + 1)

            _wait(tab_sh, idx_v, rows, sem, ch)
            _acc_chunk(rows, acc_v, k)
        pltpu.sync_copy(acc_v, out_hbm.at[pl.ds(base + g * GN, GN)])
        return carry

    lax.fori_loop(0, NG, obody, 0)


def _gather_sum(table_full, pidx_flat):
    mesh = plsc.VectorSubcoreMesh(
        core_axis_name="c", subcore_axis_name="s",
        num_cores=NC, num_subcores=NS)
    return pl.kernel(
        _gather_sum_body,
        out_type=jax.ShapeDtypeStruct((N2, MK), jnp.float32),
        mesh=mesh,
        scratch_types=[
            pltpu.VMEM((NPW * L,), jnp.int32),
            pltpu.VMEM((CH_ROWS, MK // 2), jnp.int32),
            pltpu.VMEM((CH_ROWS, MK // 2), jnp.int32),
            pltpu.VMEM((GN, MK), jnp.float32),
            pltpu.VMEM_SHARED((NT, MK // 2), jnp.int32),
            pltpu.SemaphoreType.DMA,
            pltpu.SemaphoreType.DMA,
        ],
    )(table_full, pidx_flat)


def _final_body(a_ref, b_ref, out_ref):
    d = jnp.abs(a_ref[:] - b_ref[:])
    r = lax.broadcasted_iota(jnp.int32, (MK, M), 0)
    c = lax.broadcasted_iota(jnp.int32, (MK, M), 1)
    g = ((r // K) == c).astype(jnp.float32)
    out_ref[:] = 1.0 - 0.5 * jnp.dot(d, g, preferred_element_type=jnp.float32,
                                     precision=lax.Precision.HIGHEST)


def _finalize(acc, b_flat):
    TN = 1024
    return pl.pallas_call(
        _final_body,
        grid=(N2 // TN,),
        in_specs=[
            pl.BlockSpec((TN, MK), lambda i: (i, 0)),
            pl.BlockSpec((1, MK), lambda i: (0, 0)),
        ],
        out_specs=pl.BlockSpec((TN, M), lambda i: (i, 0)),
        out_shape=jax.ShapeDtypeStruct((N2, M), jnp.float32),
    )(acc, b_flat)


def kernel(x, edge_index, packed_idx, mask, B, temp):
    del edge_index
    x_pad = jnp.pad(x, ((0, NT - N), (0, 0)))
    table = _build_table(x_pad, jnp.reshape(mask, (MK, F)), temp)
    pidx = jnp.pad(packed_idx, ((0, N2 - N), (0, 0)))
    pidx = jnp.where(pidx > 0, pidx - 1, N)
    acc = _gather_sum(table, jnp.reshape(pidx, (-1,)))
    res = _finalize(acc, jnp.reshape(B, (1, MK)))
    return res[:N]
